# baseline (speedup 1.0000x reference)
import numpy as np

import jax
import jax.numpy as jnp
from jax.experimental import pallas as pl
from jax.experimental.pallas import tpu as pltpu

_B_BLK = 256
_V_BLK = 2048

_KS0 = np.uint32(0)
_KS1 = np.uint32(42)
_KS2 = np.uint32(np.uint32(0x1BD11BDA) ^ np.uint32(42))
_ROT = ((13, 15, 26, 6), (17, 29, 16, 24))
_TINY = np.float32(np.finfo(np.float32).tiny)


def _gumbel_bits(lin_u32):
    ks = (_KS0, _KS1, _KS2)
    x0 = jnp.zeros_like(lin_u32)
    x1 = lin_u32 + _KS1

    for r in range(5):
        for d in _ROT[r % 2]:
            x0 = x0 + x1
            x1 = (x1 << np.uint32(d)) | (x1 >> np.uint32(32 - d))
            x1 = x0 ^ x1
        x0 = x0 + ks[(r + 1) % 3]
        x1 = x1 + ks[(r + 2) % 3] + np.uint32(r + 1)

    bits = x0 ^ x1
    fb = (bits >> np.uint32(9)) | np.uint32(0x3F800000)
    u = jax.lax.bitcast_convert_type(fb, jnp.float32) - jnp.float32(1.0)
    u = jnp.maximum(_TINY, u * (np.float32(1.0) - _TINY) + _TINY)
    return -jnp.log(-jnp.log(u))


def _sample_body(n_vb, v_total, logits_ref, idx_ref, best_val, best_idx):
    bb = pl.program_id(0)
    vb = pl.program_id(1)

    @pl.when(vb == 0)
    def _init():
        best_val[...] = jnp.full_like(best_val, -jnp.inf)
        best_idx[...] = jnp.zeros_like(best_idx)

    rows = jax.lax.broadcasted_iota(jnp.int32, (_B_BLK, _V_BLK), 0) + bb * _B_BLK
    cols = jax.lax.broadcasted_iota(jnp.int32, (_B_BLK, _V_BLK), 1) + vb * _V_BLK
    lin = rows * v_total + cols
    g = _gumbel_bits(lin.astype(jnp.uint32))
    s = g + logits_ref[...]
    s = jnp.where(cols < v_total, s, -jnp.inf)

    m = jnp.max(s, axis=1, keepdims=True)
    cand = jnp.where(s == m, cols, jnp.int32(2**31 - 1))
    li = jnp.min(cand, axis=1, keepdims=True)

    upd = m > best_val[...]
    best_val[...] = jnp.where(upd, m, best_val[...])
    best_idx[...] = jnp.where(upd, li, best_idx[...])

    @pl.when(vb == n_vb - 1)
    def _flush():
        idx_ref[...] = best_idx[...]


def _onehot_body(v_total, idx_ref, out_ref):
    vb = pl.program_id(1)
    cols = jax.lax.broadcasted_iota(jnp.int32, (_B_BLK, _V_BLK), 1) + vb * _V_BLK
    out_ref[...] = (cols == idx_ref[...]).astype(jnp.float32)


def kernel(logits):
    b, v = logits.shape
    n_bb = pl.cdiv(b, _B_BLK)
    n_vb = pl.cdiv(v, _V_BLK)

    idx = pl.pallas_call(
        lambda *refs: _sample_body(n_vb, v, *refs),
        grid=(n_bb, n_vb),
        in_specs=[pl.BlockSpec((_B_BLK, _V_BLK), lambda i, j: (i, j))],
        out_specs=pl.BlockSpec((_B_BLK, 1), lambda i, j: (i, 0)),
        out_shape=jax.ShapeDtypeStruct((b, 1), jnp.int32),
        scratch_shapes=[
            pltpu.VMEM((_B_BLK, 1), jnp.float32),
            pltpu.VMEM((_B_BLK, 1), jnp.int32),
        ],
        compiler_params=pltpu.CompilerParams(
            dimension_semantics=("parallel", "arbitrary"),
        ),
    )(logits)

    out = pl.pallas_call(
        lambda *refs: _onehot_body(v, *refs),
        grid=(n_bb, n_vb),
        in_specs=[pl.BlockSpec((_B_BLK, 1), lambda i, j: (i, 0))],
        out_specs=pl.BlockSpec((_B_BLK, _V_BLK), lambda i, j: (i, j)),
        out_shape=jax.ShapeDtypeStruct((b, v), jnp.float32),
        compiler_params=pltpu.CompilerParams(
            dimension_semantics=("parallel", "arbitrary"),
        ),
    )(idx)
    return out


# speedup vs baseline: 1.0107x; 1.0107x over previous
import numpy as np

import jax
import jax.numpy as jnp
from jax.experimental import pallas as pl
from jax.experimental.pallas import tpu as pltpu

_B_BLK = 256
_V_BLK = 2048

_KS0 = np.uint32(0)
_KS1 = np.uint32(42)
_KS2 = np.uint32(np.uint32(0x1BD11BDA) ^ np.uint32(42))
_ROT = ((13, 15, 26, 6), (17, 29, 16, 24))
_TINY = np.float32(np.finfo(np.float32).tiny)


def _gumbel_bits(lin_u32):
    ks = (_KS0, _KS1, _KS2)
    x0 = jnp.zeros_like(lin_u32)
    x1 = lin_u32 + _KS1

    for r in range(5):
        for d in _ROT[r % 2]:
            x0 = x0 + x1
            x1 = (x1 << np.uint32(d)) | (x1 >> np.uint32(32 - d))
            x1 = x0 ^ x1
        x0 = x0 + ks[(r + 1) % 3]
        x1 = x1 + ks[(r + 2) % 3] + np.uint32(r + 1)

    bits = x0 ^ x1
    fb = (bits >> np.uint32(9)) | np.uint32(0x3F800000)
    u = jax.lax.bitcast_convert_type(fb, jnp.float32) - jnp.float32(1.0)
    u = jnp.maximum(_TINY, u * (np.float32(1.0) - _TINY) + _TINY)
    return -jnp.log(-jnp.log(u))


def _sample_body(n_vb, v_total, logits_ref, idx_ref, best_val, best_idx):
    bb = pl.program_id(0)
    vb = pl.program_id(1)

    @pl.when(vb == 0)
    def _init():
        best_val[...] = jnp.full_like(best_val, -jnp.inf)
        best_idx[...] = jnp.zeros_like(best_idx)

    rows = jax.lax.broadcasted_iota(jnp.int32, (_B_BLK, _V_BLK), 0) + bb * _B_BLK
    cols = jax.lax.broadcasted_iota(jnp.int32, (_B_BLK, _V_BLK), 1) + vb * _V_BLK
    lin = rows * v_total + cols
    g = _gumbel_bits(lin.astype(jnp.uint32))
    s = g + logits_ref[...]
    s = jnp.where(cols < v_total, s, -jnp.inf)

    m = jnp.max(s, axis=1, keepdims=True)
    cand = jnp.where(s == m, cols, jnp.int32(2**31 - 1))
    li = jnp.min(cand, axis=1, keepdims=True)

    upd = m > best_val[...]
    best_val[...] = jnp.where(upd, m, best_val[...])
    best_idx[...] = jnp.where(upd, li, best_idx[...])

    @pl.when(vb == n_vb - 1)
    def _flush():
        idx_ref[...] = best_idx[...]


_OH_B_BLK = 256
_OH_V_BLK = 8192


def _onehot_body(v_total, idx_ref, out_ref):
    vb = pl.program_id(1)
    cols = (
        jax.lax.broadcasted_iota(jnp.int32, (_OH_B_BLK, _OH_V_BLK), 1)
        + vb * _OH_V_BLK
    )
    out_ref[...] = (cols == idx_ref[...]).astype(jnp.float32)


def kernel(logits):
    b, v = logits.shape
    n_bb = pl.cdiv(b, _B_BLK)
    n_vb = pl.cdiv(v, _V_BLK)

    idx = pl.pallas_call(
        lambda *refs: _sample_body(n_vb, v, *refs),
        grid=(n_bb, n_vb),
        in_specs=[pl.BlockSpec((_B_BLK, _V_BLK), lambda i, j: (i, j))],
        out_specs=pl.BlockSpec((_B_BLK, 1), lambda i, j: (i, 0)),
        out_shape=jax.ShapeDtypeStruct((b, 1), jnp.int32),
        scratch_shapes=[
            pltpu.VMEM((_B_BLK, 1), jnp.float32),
            pltpu.VMEM((_B_BLK, 1), jnp.int32),
        ],
        compiler_params=pltpu.CompilerParams(
            dimension_semantics=("parallel", "arbitrary"),
        ),
    )(logits)

    out = pl.pallas_call(
        lambda *refs: _onehot_body(v, *refs),
        grid=(pl.cdiv(b, _OH_B_BLK), pl.cdiv(v, _OH_V_BLK)),
        in_specs=[pl.BlockSpec((_OH_B_BLK, 1), lambda i, j: (i, 0))],
        out_specs=pl.BlockSpec((_OH_B_BLK, _OH_V_BLK), lambda i, j: (i, j)),
        out_shape=jax.ShapeDtypeStruct((b, v), jnp.float32),
        compiler_params=pltpu.CompilerParams(
            dimension_semantics=("parallel", "arbitrary"),
        ),
    )(idx)
    return out
